# Initial kernel scaffold; baseline (speedup 1.0000x reference)
import jax
import jax.numpy as jnp
from jax.experimental import pallas as pl


def kernel(x, mask, emb):
    raise NotImplementedError("write your pallas kernel here")


# speedup vs baseline: 1.3515x; 1.3515x over previous
import functools

import jax
import jax.numpy as jnp
from jax import lax
from jax.experimental import pallas as pl
from jax.experimental.pallas import tpu as pltpu
from jax.experimental.pallas import tpu_sc as plsc

VOCAB_ROWS = 8021
DIM = 312
NUM_IDX = 4096 * 200

NUM_CORES = 2
NUM_SUBCORES = 16
NUM_WORKERS = NUM_CORES * NUM_SUBCORES

CHUNK = 128
ROWS_PER_WORKER = NUM_IDX // NUM_WORKERS
CHUNKS_PER_WORKER = ROWS_PER_WORKER // CHUNK


def _gather_body(table_hbm, idx_hbm, out_hbm, idx_v, rows_v, sem):
    wid = lax.axis_index("s") * NUM_CORES + lax.axis_index("c")
    worker_base = wid * ROWS_PER_WORKER

    def body(c, carry):
        base = worker_base + c * CHUNK
        pltpu.sync_copy(idx_hbm.at[pl.ds(base, CHUNK)], idx_v)
        pltpu.async_copy(table_hbm.at[idx_v], rows_v, sem).wait()
        pltpu.sync_copy(rows_v, out_hbm.at[pl.ds(base, CHUNK)])
        return carry

    lax.fori_loop(0, CHUNKS_PER_WORKER, body, 0)


@jax.jit
def _embedding_gather(emb, idx):
    mesh = plsc.VectorSubcoreMesh(core_axis_name="c", subcore_axis_name="s")
    run = functools.partial(
        pl.kernel,
        out_type=jax.ShapeDtypeStruct((NUM_IDX, DIM), jnp.float32),
        mesh=mesh,
        scratch_types=[
            pltpu.VMEM((CHUNK,), jnp.int32),
            pltpu.VMEM((CHUNK, DIM), jnp.float32),
            pltpu.SemaphoreType.DMA,
        ],
        compiler_params=pltpu.CompilerParams(use_tc_tiling_on_sc=False),
    )(_gather_body)
    return run(emb, idx)


def kernel(x, mask, emb):
    idx = x.reshape(-1).astype(jnp.int32)
    flat = _embedding_gather(emb, idx)
    y = flat.reshape(x.shape[0], x.shape[1], DIM)
    return (y, y)


# speedup vs baseline: 1.4304x; 1.0584x over previous
import functools

import jax
import jax.numpy as jnp
from jax import lax
from jax.experimental import pallas as pl
from jax.experimental.pallas import tpu as pltpu
from jax.experimental.pallas import tpu_sc as plsc

VOCAB_ROWS = 8021
DIM = 312
NUM_IDX = 4096 * 200

NUM_CORES = 2
NUM_SUBCORES = 16
NUM_WORKERS = NUM_CORES * NUM_SUBCORES

CHUNK = 80
NBUF = 4
ROWS_PER_WORKER = NUM_IDX // NUM_WORKERS
CHUNKS_PER_WORKER = ROWS_PER_WORKER // CHUNK
LOOKAHEAD = 2


def _gather_body(table_hbm, idx_hbm, out_hbm, idx_v, rows, gsems, wsems):
    wid = lax.axis_index("s") * NUM_CORES + lax.axis_index("c")
    row_base = wid * ROWS_PER_WORKER
    chunk_base = wid * CHUNKS_PER_WORKER

    pltpu.sync_copy(idx_hbm.at[pl.ds(chunk_base, CHUNKS_PER_WORKER)], idx_v)

    def start_gather(c, b):
        pltpu.async_copy(table_hbm.at[idx_v.at[c]], rows.at[b], gsems.at[b])

    def wait_gather(b):
        pltpu.make_async_copy(table_hbm.at[idx_v.at[0]], rows.at[b],
                              gsems.at[b]).wait()

    def start_write(c, b):
        pltpu.async_copy(rows.at[b], out_hbm.at[pl.ds(row_base + c * CHUNK, CHUNK)],
                         wsems.at[b])

    def wait_write(b):
        pltpu.make_async_copy(rows.at[b], out_hbm.at[pl.ds(row_base, CHUNK)],
                              wsems.at[b]).wait()

    for c in range(LOOKAHEAD):
        start_gather(c, c % NBUF)

    def step(s, carry):
        for i in range(NBUF):
            c = s * NBUF + i
            b = c % NBUF
            bf = (c + LOOKAHEAD) % NBUF
            cf = c + LOOKAHEAD

            @pl.when(cf < CHUNKS_PER_WORKER)
            def _():
                @pl.when(cf >= NBUF)
                def _():
                    wait_write(bf)
                start_gather(cf, bf)

            wait_gather(b)
            start_write(c, b)
        return carry

    lax.fori_loop(0, CHUNKS_PER_WORKER // NBUF, step, 0)

    for b in range(NBUF):
        wait_write(b)


@jax.jit
def _embedding_gather(emb, idx2d):
    mesh = plsc.VectorSubcoreMesh(core_axis_name="c", subcore_axis_name="s")
    run = functools.partial(
        pl.kernel,
        out_type=jax.ShapeDtypeStruct((NUM_IDX, DIM), jnp.float32),
        mesh=mesh,
        scratch_types=[
            pltpu.VMEM((CHUNKS_PER_WORKER, CHUNK), jnp.int32),
            pltpu.VMEM((NBUF, CHUNK, DIM), jnp.float32),
            pltpu.SemaphoreType.DMA((NBUF,)),
            pltpu.SemaphoreType.DMA((NBUF,)),
        ],
        compiler_params=pltpu.CompilerParams(use_tc_tiling_on_sc=False),
    )(_gather_body)
    return run(emb, idx2d)


def kernel(x, mask, emb):
    idx2d = x.reshape(NUM_IDX // CHUNK, CHUNK).astype(jnp.int32)
    flat = _embedding_gather(emb, idx2d)
    y = flat.reshape(x.shape[0], x.shape[1], DIM)
    return (y, y)


# speedup vs baseline: 2.2415x; 1.5671x over previous
import functools

import jax
import jax.numpy as jnp
from jax import lax
from jax.experimental import pallas as pl
from jax.experimental.pallas import tpu as pltpu
from jax.experimental.pallas import tpu_sc as plsc

VOCAB_ROWS = 8021
DIM = 312
DIM_PAD = 384
NUM_IDX = 4096 * 200

NUM_CORES = 2
NUM_SUBCORES = 16
NUM_WORKERS = NUM_CORES * NUM_SUBCORES

CHUNK = 64
NBUF = 4
ROWS_PER_WORKER = NUM_IDX // NUM_WORKERS
CHUNKS_PER_WORKER = ROWS_PER_WORKER // CHUNK
LOOKAHEAD = 2

STRIPS = ((0, 128), (128, 128), (256, DIM - 256))


def _gather_body(table_hbm, idx_hbm, out_hbm, idx_bufs, rows, isems, gsems,
                 wsems):
    wid = lax.axis_index("s") * NUM_CORES + lax.axis_index("c")
    row_base = wid * ROWS_PER_WORKER

    def start_idx(c, b):
        pltpu.async_copy(idx_hbm.at[pl.ds(row_base + c * CHUNK, CHUNK)],
                         idx_bufs.at[b], isems.at[b])

    def wait_idx(b):
        pltpu.make_async_copy(idx_hbm.at[pl.ds(row_base, CHUNK)],
                              idx_bufs.at[b], isems.at[b]).wait()

    def start_gather(b):
        pltpu.async_copy(table_hbm.at[idx_bufs.at[b]], rows.at[b], gsems.at[b])

    def wait_gather(b):
        pltpu.make_async_copy(table_hbm.at[idx_bufs.at[0]], rows.at[b],
                              gsems.at[b]).wait()

    def start_write(c, b):
        pltpu.async_copy(rows.at[b],
                         out_hbm.at[pl.ds(row_base + c * CHUNK, CHUNK)],
                         wsems.at[b])

    def wait_write(b):
        pltpu.make_async_copy(rows.at[b],
                              out_hbm.at[pl.ds(row_base, CHUNK)],
                              wsems.at[b]).wait()

    for c in range(LOOKAHEAD):
        start_idx(c, c % NBUF)
    for c in range(LOOKAHEAD):
        wait_idx(c % NBUF)
        start_gather(c % NBUF)

    def step(s, carry):
        for i in range(NBUF):
            c = s * NBUF + i
            b = c % NBUF
            cf = c + LOOKAHEAD
            bf = cf % NBUF

            @pl.when(cf < CHUNKS_PER_WORKER)
            def _():
                @pl.when(cf >= NBUF)
                def _():
                    wait_write(bf)
                start_idx(cf, bf)
                wait_idx(bf)
                start_gather(bf)

            wait_gather(b)
            start_write(c, b)
        return carry

    lax.fori_loop(0, CHUNKS_PER_WORKER // NBUF, step, 0)

    for b in range(NBUF):
        wait_write(b)


@jax.jit
def _embedding_gather(table, idx):
    mesh = plsc.VectorSubcoreMesh(core_axis_name="c", subcore_axis_name="s")
    run = functools.partial(
        pl.kernel,
        out_type=jax.ShapeDtypeStruct((NUM_IDX, DIM_PAD), jnp.float32),
        mesh=mesh,
        scratch_types=[
            pltpu.VMEM((NBUF, CHUNK), jnp.int32),
            pltpu.VMEM((NBUF, CHUNK, DIM_PAD), jnp.float32),
            pltpu.SemaphoreType.DMA((NBUF,)),
            pltpu.SemaphoreType.DMA((NBUF,)),
            pltpu.SemaphoreType.DMA((NBUF,)),
        ],
    )(_gather_body)
    return run(table, idx)


def kernel(x, mask, emb):
    idx = x.reshape(-1).astype(jnp.int32)
    table = jnp.pad(emb, ((0, 0), (0, DIM_PAD - DIM)))
    flat = _embedding_gather(table, idx)
    y = flat[:, :DIM].reshape(x.shape[0], x.shape[1], DIM)
    return (y, y)
